# baseline (speedup 1.0000x reference)
import functools
import math

import jax
import jax.numpy as jnp
from jax import lax
from jax.experimental import pallas as pl
from jax.experimental.pallas import tpu as pltpu
from jax.experimental.pallas import tpu_sc as plsc

SCALE = 32.0
ALPHA = 0.2
RBLK = 8


def _gather_targets(y32, score_flat, batch, num_cls):
    info = plsc.get_sparse_core_info()
    nw = info.num_cores * info.num_subcores
    bpw = batch // nw

    mesh = plsc.VectorSubcoreMesh(core_axis_name="c", subcore_axis_name="s")

    @functools.partial(
        pl.kernel,
        mesh=mesh,
        out_type=jax.ShapeDtypeStruct((batch,), jnp.float32),
        scratch_types=[
            pltpu.VMEM((bpw,), jnp.int32),
            pltpu.VMEM((bpw,), jnp.int32),
            pltpu.VMEM((bpw,), jnp.float32),
            pltpu.SemaphoreType.DMA,
        ],
    )
    def k(y_hbm, flat_hbm, out_hbm, y_v, idx_v, vals_v, sem):
        wid = lax.axis_index("s") * info.num_cores + lax.axis_index("c")
        base = wid * bpw
        pltpu.sync_copy(y_hbm.at[pl.ds(base, bpw)], y_v)
        for c in range(bpw // 16):
            rows = base + c * 16 + lax.iota(jnp.int32, 16)
            idx_v[pl.ds(c * 16, 16)] = y_v[pl.ds(c * 16, 16)] + rows * num_cls
        pltpu.async_copy(flat_hbm.at[idx_v], vals_v, sem).wait()
        pltpu.sync_copy(vals_v, out_hbm.at[pl.ds(base, bpw)])

    return k(y32, score_flat)


def _dense_loss(score, t_col, batch, num_cls):
    corr = math.exp(-SCALE * ALPHA) - 1.0
    nbuf = 4
    steps = batch // RBLK

    def body(t_ref, score_hbm, out_ref, bufs, sems):
        def start(i, slot):
            pltpu.make_async_copy(
                score_hbm.at[pl.ds(i * RBLK, RBLK), :],
                bufs.at[slot],
                sems.at[slot],
            ).start()

        for i in range(nbuf):
            start(i, i)

        def step_fn(i, carry):
            slot = lax.rem(i, nbuf)
            pltpu.make_async_copy(
                score_hbm.at[pl.ds(i * RBLK, RBLK), :],
                bufs.at[slot],
                sems.at[slot],
            ).wait()
            v = bufs[slot] * SCALE
            m = jnp.max(v, axis=1, keepdims=True)
            s = jnp.sum(jnp.exp(v - m), axis=1, keepdims=True)
            tt = t_ref[pl.ds(i * RBLK, RBLK), :] * SCALE
            out_ref[pl.ds(i * RBLK, RBLK), :] = (
                m + jnp.log(s + jnp.exp(tt - m) * corr) - tt + SCALE * ALPHA
            )
            nxt = i + nbuf

            @pl.when(nxt < steps)
            def _():
                start_slot = lax.rem(nxt, nbuf)
                pltpu.make_async_copy(
                    score_hbm.at[pl.ds(nxt * RBLK, RBLK), :],
                    bufs.at[start_slot],
                    sems.at[start_slot],
                ).start()

            return carry

        lax.fori_loop(0, steps, step_fn, 0)

    return pl.pallas_call(
        body,
        in_specs=[
            pl.BlockSpec(memory_space=pltpu.VMEM),
            pl.BlockSpec(memory_space=pltpu.HBM),
        ],
        out_specs=pl.BlockSpec(memory_space=pltpu.VMEM),
        out_shape=jax.ShapeDtypeStruct((batch, 1), jnp.float32),
        scratch_shapes=[
            pltpu.VMEM((nbuf, RBLK, num_cls), jnp.float32),
            pltpu.SemaphoreType.DMA((nbuf,)),
        ],
    )(t_col, score)


def kernel(score, y):
    batch, num_cls = score.shape
    y32 = jnp.asarray(y).reshape(-1).astype(jnp.int32)
    t = _gather_targets(y32, score.reshape(-1), batch, num_cls)
    out = _dense_loss(score, t.reshape(batch, 1), batch, num_cls)
    return out[:, 0]
